# baseline (speedup 1.0000x reference)
import functools

import jax
import jax.numpy as jnp
from jax import lax
from jax.experimental import pallas as pl
from jax.experimental.pallas import tpu as pltpu
from jax.experimental.pallas import tpu_sc as plsc

NP_, NJ_, NB_, NA_ = 40000, 30000, 20000, 10000
HID = 20
NC, NS, LANE = 2, 16, 128
WORKERS = NC * NS

SIZES = {'p': NP_, 'j': NJ_, 'b': NB_, 'a': NA_}
RELS = [('p', 'p'), ('j', 'j'), ('b', 'b'), ('p', 'j'),
        ('j', 'p'), ('b', 'a'), ('a', 'b'), ('p', 'a')]
E_REL = 64000

NI = 48
E_CALL = WORKERS * NI * LANE
TROWS = 80000
ACC = 70016
SINK = 70000

GROUPS = [[0, 1], [2, 3, 5], [4, 6, 7]]

_mesh = plsc.VectorSubcoreMesh(core_axis_name="c", subcore_axis_name="s")


@functools.partial(
    pl.kernel, mesh=_mesh,
    out_type=jax.ShapeDtypeStruct((NC, ACC, HID), jnp.float32),
    scratch_types=[
        pltpu.VMEM((NI, LANE), jnp.int32),
        pltpu.VMEM((NI, LANE), jnp.int32),
        pltpu.VMEM((LANE, HID), jnp.float32),
        pltpu.VMEM_SHARED((ACC, HID), jnp.float32),
        pltpu.SemaphoreType.DMA,
    ],
    compiler_params=pltpu.CompilerParams(use_tc_tiling_on_sc=False),
)
def _sc_agg(table, srcg, dstg, zeros, out, src_v, dst_v, rows_v, acc_sh, sem):
    zch = ACC // NS
    cid = lax.axis_index("c")
    sid = lax.axis_index("s")
    pltpu.sync_copy(zeros, acc_sh.at[pl.ds(sid * zch, zch)])
    plsc.subcore_barrier()
    pltpu.sync_copy(srcg.at[cid, sid], src_v)
    pltpu.sync_copy(dstg.at[cid, sid], dst_v)

    def body(j, carry):
        pltpu.async_copy(table.at[src_v.at[j]], rows_v, sem).wait()
        pltpu.sync_copy(rows_v, acc_sh.at[dst_v.at[j]], add=True)
        return carry

    lax.fori_loop(0, NI, body, 0)
    plsc.subcore_barrier()
    pltpu.sync_copy(acc_sh.at[pl.ds(sid * zch, zch)],
                    out.at[cid, pl.ds(sid * zch, zch)])


_ZEROS = None


def _split_edges(idx, pad_value):
    pad = E_CALL - idx.shape[0]
    if pad:
        idx = jnp.concatenate([idx, jnp.full((pad,), pad_value, jnp.int32)])
    return idx.reshape(NC, NS, NI, LANE)


def _agg(table, srcg, dstg):
    if table.shape[0] != TROWS:
        table = jnp.pad(table, ((0, TROWS - table.shape[0]), (0, 0)))
    zeros = jnp.zeros((ACC // NS, HID), jnp.float32)
    out = _sc_agg(table, srcg, dstg, zeros)
    return out[0] + out[1]


_BM = 2048


def _mm_body(x_ref, w_ref, o_ref):
    o_ref[...] = jnp.dot(x_ref[...], w_ref[...],
                         preferred_element_type=jnp.float32)


def _mm(x, w):
    m, k = x.shape
    n = w.shape[1]
    mp = ((m + _BM - 1) // _BM) * _BM
    if mp != m:
        x = jnp.pad(x, ((0, mp - m), (0, 0)))
    out = pl.pallas_call(
        _mm_body,
        grid=(mp // _BM,),
        in_specs=[pl.BlockSpec((_BM, k), lambda i: (i, 0)),
                  pl.BlockSpec((k, n), lambda i: (0, 0))],
        out_specs=pl.BlockSpec((_BM, n), lambda i: (i, 0)),
        out_shape=jax.ShapeDtypeStruct((mp, n), jnp.float32),
    )(x, w)
    return out[:m]


def _pad20(h):
    if h.shape[1] == HID:
        return h
    return jnp.pad(h, ((0, 0), (0, HID - h.shape[1])))


def _hetero_layer(feats, W, b, ctx):
    dh = W.shape[2]
    by_src = {'p': [0, 3, 7], 'j': [1, 4], 'b': [2, 5], 'a': [6]}
    h = {}
    for t, rels in by_src.items():
        wcat = jnp.concatenate([W[r] for r in rels], axis=1)
        ht = _mm(feats[t], wcat)
        for i, r in enumerate(rels):
            h[r] = _pad20(ht[:, i * dh:(i + 1) * dh] * ctx['so'][r])
    out = {t: 0.0 for t in SIZES}
    for gi, group in enumerate(GROUPS):
        table = jnp.concatenate([h[r] for r in group], axis=0)
        acc = _agg(table, *ctx['het_edges'][gi])
        off = 0
        for r in group:
            d = RELS[r][1]
            nd = SIZES[d]
            out[d] = out[d] + acc[off:off + nd, :dh] * ctx['si'][r]
            off += nd
    for t in SIZES:
        bias = sum(b[r] for r in range(8) if RELS[r][1] == t)
        out[t] = out[t] + bias[None, :]
    return out


def _rgcn(feats, p, name, ctx):
    h = _hetero_layer(feats, p[name + '_W1'], p[name + '_b1'], ctx)
    h = {k: jax.nn.relu(v) for k, v in h.items()}
    return _hetero_layer(h, p[name + '_W2'], p[name + '_b2'], ctx)


def _sage_conv(xs, p, names, suffix, ctx):
    out = {}
    for t in ('p', 'j', 'b'):
        nm = names[t]
        wcat = jnp.concatenate([p[nm + '_Ws' + suffix],
                                p[nm + '_Wn' + suffix]], axis=1)
        proj = _mm(xs[t], wcat)
        dh = p[nm + '_Ws' + suffix].shape[1]
        table = proj[:, dh:]
        n = SIZES[t]
        agg = 0.0
        for srcg, dstg in ctx['sage_edges'][t]:
            agg = agg + _agg(table, srcg, dstg)[:n]
        out[t] = (proj[:, :dh] + agg * ctx['sage_inv_deg'][t]
                  + p[nm + '_b' + suffix][None, :])
    return out


def _sage_block(xs, p, names, ctx):
    h = _sage_conv(xs, p, names, '1', ctx)
    h = {t: jax.nn.relu(v) for t, v in h.items()}
    return _sage_conv(h, p, names, '2', ctx)


def kernel(x_power, x_junc, x_bs, x_aoi, gmnn, dp_coupled, dp_elec, dp_junc, dp_bs, hetero_edges, edge_elec, edge_junc, edge_bs, ce_W1, ce_b1, ce_W2, ce_b2, ee_Ws1, ee_Wn1, ee_b1, ee_Ws2, ee_Wn2, ee_b2, je_Ws1, je_Wn1, je_b1, je_Ws2, je_Wn2, je_b2, be_Ws1, be_Wn1, be_b1, be_Ws2, be_Wn2, be_b2, l1_W1, l1_b1, l1_W2, l1_b2, cd_W1, cd_b1, cd_W2, cd_b2, ed_Ws1, ed_Wn1, ed_b1, ed_Ws2, ed_Wn2, ed_b2, jd_Ws1, jd_Wn1, jd_b1, jd_Ws2, jd_Wn2, jd_b2, bd_Ws1, bd_Wn1, bd_b1, bd_Ws2, bd_Wn2, bd_b2, l2_W1, l2_b1, l2_W2, l2_b2, l3_W1, l3_b1, l3_W2, l3_b2):
    p = dict(locals())

    het_edges = []
    for group in GROUPS:
        src_off, dst_off = [0], [0]
        for r in group[:-1]:
            src_off.append(src_off[-1] + SIZES[RELS[r][0]])
            dst_off.append(dst_off[-1] + SIZES[RELS[r][1]])
        src = jnp.concatenate(
            [hetero_edges[r, 0] + src_off[i] for i, r in enumerate(group)])
        dst = jnp.concatenate(
            [hetero_edges[r, 1] + dst_off[i] for i, r in enumerate(group)])
        het_edges.append((_split_edges(src, 0), _split_edges(dst, SINK)))

    sage_edges = {}
    for t, ei in (('p', edge_elec), ('j', edge_junc), ('b', edge_bs)):
        chunks = []
        e = ei.shape[1]
        n_call = -(-e // E_CALL)
        step = e // n_call
        for c in range(n_call):
            lo, hi = c * step, (c + 1) * step if c < n_call - 1 else e
            chunks.append((_split_edges(ei[0, lo:hi], 0),
                           _split_edges(ei[1, lo:hi], SINK)))
        sage_edges[t] = chunks

    segs_o = [('o', r, hetero_edges[r, 0], SIZES[RELS[r][0]])
              for r in range(8)]
    segs_i = [('i', r, hetero_edges[r, 1], SIZES[RELS[r][1]])
              for r in range(8)]
    segs_s = []
    for t, ei in (('p', edge_elec), ('j', edge_junc), ('b', edge_bs)):
        dst = ei[1]
        e = dst.shape[0]
        n_call = -(-e // E_CALL)
        step = e // n_call
        for c in range(n_call):
            lo, hi = c * step, (c + 1) * step if c < n_call - 1 else e
            segs_s.append(('s', t, dst[lo:hi], SIZES[t]))
    packs, cur, cur_rows, cur_e = [], [], 0, 0
    for seg in segs_o + segs_i + segs_s:
        rows, ecnt = seg[3], seg[2].shape[0]
        if cur and (cur_rows + rows > SINK or cur_e + ecnt > E_CALL):
            packs.append(cur)
            cur, cur_rows, cur_e = [], 0, 0
        cur.append((seg, cur_rows))
        cur_rows += rows
        cur_e += ecnt
    packs.append(cur)

    ones_table = jnp.ones((TROWS, HID), jnp.float32)
    deg_o, deg_i, deg_s = {}, {}, {}
    for pack in packs:
        idx = jnp.concatenate([seg[2] + off for seg, off in pack])
        counts = _agg(ones_table, _split_edges(jnp.zeros_like(idx), 0),
                      _split_edges(idx, SINK))[:, 0]
        for (kind, key, _, rows), off in pack:
            c = counts[off:off + rows]
            if kind == 'o':
                deg_o[key] = c
            elif kind == 'i':
                deg_i[key] = c
            else:
                deg_s[key] = deg_s.get(key, 0.0) + c

    so = {r: lax.rsqrt(jnp.clip(deg_o[r], 1.0))[:, None] for r in range(8)}
    si = {r: lax.rsqrt(jnp.clip(deg_i[r], 1.0))[:, None] for r in range(8)}
    sage_inv_deg = {t: (1.0 / jnp.clip(deg_s[t], 1.0))[:, None]
                    for t in ('p', 'j', 'b')}

    ctx = dict(het_edges=het_edges, sage_edges=sage_edges, so=so, si=si,
               sage_inv_deg=sage_inv_deg)

    xp = jnp.concatenate([x_power, gmnn[:NP_], dp_coupled[:NP_], dp_elec], 1)
    xj = jnp.concatenate([x_junc, gmnn[NP_:NP_ + NJ_],
                          dp_coupled[NP_:NP_ + NJ_], dp_junc], 1)
    xb = jnp.concatenate([x_bs, gmnn[NP_ + NJ_:NP_ + NJ_ + NB_],
                          dp_coupled[NP_ + NJ_:NP_ + NJ_ + NB_], dp_bs], 1)
    xa = jnp.concatenate([x_aoi, gmnn[NP_ + NJ_ + NB_:],
                          dp_coupled[NP_ + NJ_ + NB_:],
                          jnp.zeros((NA_, 1), jnp.float32)], 1)
    feats = {'p': xp, 'j': xj, 'b': xb, 'a': xa}

    cz = _rgcn(feats, p, 'ce', ctx)
    enc = _sage_block({'p': xp, 'j': xj, 'b': xb}, p,
                      {'p': 'ee', 'j': 'je', 'b': 'be'}, ctx)
    cz['a'] = jnp.concatenate([cz['a'], jnp.zeros((NA_, HID), jnp.float32)], 1)
    for t in ('p', 'j', 'b'):
        cz[t] = jnp.concatenate([cz[t], enc[t]], 1)
    cz = _rgcn(cz, p, 'l1', ctx)
    dz = _rgcn(cz, p, 'cd', ctx)
    dec = _sage_block({'p': cz['p'], 'j': cz['j'], 'b': cz['b']}, p,
                      {'p': 'ed', 'j': 'jd', 'b': 'bd'}, ctx)
    for t in ('p', 'j', 'b'):
        dz[t] = jnp.concatenate([dz[t], dec[t]], 1)
    dz['a'] = jnp.concatenate([dz['a'], jnp.zeros((NA_, HID), jnp.float32)], 1)
    dz = _rgcn(dz, p, 'l2', ctx)
    dz = {k: jax.nn.relu(v) for k, v in dz.items()}
    dz = _rgcn(dz, p, 'l3', ctx)
    return tuple(jax.nn.softmax(dz[t], axis=1) for t in ('p', 'j', 'b', 'a'))
